# initial kernel scaffold (unmeasured)
import jax
import jax.numpy as jnp
from jax import lax
from jax.experimental import pallas as pl
from jax.experimental.pallas import tpu as pltpu


def kernel(
    x,
):
    def body(*refs):
        pass

    out_shape = jax.ShapeDtypeStruct(..., jnp.float32)
    return pl.pallas_call(body, out_shape=out_shape)(...)


# baseline (device time: 29570 ns/iter reference)
import functools

import jax
import jax.numpy as jnp
from jax import lax
from jax.experimental import pallas as pl
from jax.experimental.pallas import tpu as pltpu

N_DEV = 8
GROUP = 128


def kernel(x):
    m, n = x.shape
    n_groups = m // GROUP

    def body(x_ref, out_ref, comm_ref, send_sems, recv_sems):
        my_pos = lax.axis_index("i")
        left = lax.rem(my_pos - 1 + N_DEV, N_DEV)
        right = lax.rem(my_pos + 1, N_DEV)

        barrier_sem = pltpu.get_barrier_semaphore()
        for nbr in (left, right):
            pl.semaphore_signal(
                barrier_sem, inc=1,
                device_id=(nbr,), device_id_type=pl.DeviceIdType.MESH,
            )
        pl.semaphore_wait(barrier_sem, 2)

        carry = jnp.ones((1, n), jnp.float32)
        for g in range(n_groups):
            v = x_ref[pl.ds(g * GROUP, GROUP), :]
            d = 1
            while d < GROUP:
                shifted = jnp.concatenate(
                    [jnp.ones((d, n), jnp.float32), v[: GROUP - d]], axis=0
                )
                v = v * shifted
                d *= 2
            v = v * carry
            out_ref[pl.ds(g * GROUP, GROUP), :] = v
            carry = v[GROUP - 1 : GROUP, :]

        comm_ref[pl.ds(0, 1), :] = carry

        for h in range(N_DEV - 1):
            rdma = pltpu.make_async_remote_copy(
                src_ref=comm_ref.at[pl.ds(h, 1)],
                dst_ref=comm_ref.at[pl.ds(h + 1, 1)],
                send_sem=send_sems.at[h],
                recv_sem=recv_sems.at[h],
                device_id=(right,),
                device_id_type=pl.DeviceIdType.MESH,
            )
            rdma.start()
            rdma.wait()

        row = lax.broadcasted_iota(jnp.int32, (N_DEV, n), 0)
        mask = (row >= 1) & (row <= my_pos)
        t = jnp.where(mask, comm_ref[...], jnp.ones((N_DEV, n), jnp.float32))
        t = t[0:4] * t[4:8]
        t = t[0:2] * t[2:4]
        prefix = t[0:1] * t[1:2]

        for g in range(n_groups):
            out_ref[pl.ds(g * GROUP, GROUP), :] = (
                out_ref[pl.ds(g * GROUP, GROUP), :] * prefix
            )

        @functools.partial(
            pl.run_scoped, second_barrier=pltpu.SemaphoreType.REGULAR
        )
        def _(second_barrier):
            for nbr in (left, right):
                pl.semaphore_signal(
                    second_barrier, inc=1,
                    device_id=(nbr,), device_id_type=pl.DeviceIdType.MESH,
                )
            pl.semaphore_wait(second_barrier, 2)

    return pl.pallas_call(
        body,
        out_shape=jax.ShapeDtypeStruct((m, n), jnp.float32),
        in_specs=[pl.BlockSpec(memory_space=pltpu.VMEM)],
        out_specs=pl.BlockSpec(memory_space=pltpu.VMEM),
        scratch_shapes=[
            pltpu.VMEM((N_DEV, n), jnp.float32),
            pltpu.SemaphoreType.DMA((N_DEV - 1,)),
            pltpu.SemaphoreType.DMA((N_DEV - 1,)),
        ],
        compiler_params=pltpu.CompilerParams(collective_id=0),
    )(x)


# device time: 20094 ns/iter; 1.4716x vs baseline; 1.4716x over previous
import functools

import jax
import jax.numpy as jnp
from jax import lax
from jax.experimental import pallas as pl
from jax.experimental.pallas import tpu as pltpu

N_DEV = 8
GROUP = 128
N_GROUPS = 2048 // GROUP
STEP_DISTS = (1, 2, 4)
GROUPS_PER_STEP = (5, 5, 6)


def kernel(x):
    m, n = x.shape

    def body(x_ref, out_ref, send_buf, recv_buf, send_sems, recv_sems):
        my_pos = lax.axis_index("i")

        barrier_sem = pltpu.get_barrier_semaphore()
        for off in range(1, N_DEV):
            pl.semaphore_signal(
                barrier_sem, inc=1,
                device_id=(lax.rem(my_pos + off, N_DEV),),
                device_id_type=pl.DeviceIdType.MESH,
            )
        pl.semaphore_wait(barrier_sem, N_DEV - 1)

        ones_row = jnp.ones((1, n), jnp.float32)

        gts = []
        for g in range(N_GROUPS):
            u = x_ref[pl.ds(g * GROUP, GROUP), :]
            r = GROUP
            while r > 1:
                u = u[: r // 2] * u[r // 2 : r]
                r //= 2
            gts.append(u)
        gps = [ones_row]
        for g in range(1, N_GROUPS):
            gps.append(gps[g - 1] * gts[g - 1])
        t_local = gps[-1] * gts[-1]

        def scan_group(g):
            v = x_ref[pl.ds(g * GROUP, GROUP), :]
            d = 1
            while d < GROUP:
                shifted = jnp.concatenate(
                    [jnp.ones((d, n), jnp.float32), v[: GROUP - d]], axis=0
                )
                v = v * shifted
                d *= 2
            out_ref[pl.ds(g * GROUP, GROUP), :] = v * gps[g]

        pre = ones_row
        descs = []
        g_next = 0
        for k, d in enumerate(STEP_DISTS):
            send_buf[pl.ds(k, 1), :] = pre * t_local
            rdma = pltpu.make_async_remote_copy(
                src_ref=send_buf.at[pl.ds(k, 1)],
                dst_ref=recv_buf.at[pl.ds(k, 1)],
                send_sem=send_sems.at[k],
                recv_sem=recv_sems.at[k],
                device_id=(lax.rem(my_pos + d, N_DEV),),
                device_id_type=pl.DeviceIdType.MESH,
            )
            descs.append(rdma)

            @pl.when(my_pos + d < N_DEV)
            def _():
                rdma.start()

            for g in range(g_next, g_next + GROUPS_PER_STEP[k]):
                scan_group(g)
            g_next += GROUPS_PER_STEP[k]

            @pl.when(my_pos >= d)
            def _():
                rdma.wait_recv()

            got = jnp.where(my_pos >= d, recv_buf[pl.ds(k, 1), :], ones_row)
            pre = pre * got

        for g in range(g_next, N_GROUPS):
            scan_group(g)

        for g in range(N_GROUPS):
            out_ref[pl.ds(g * GROUP, GROUP), :] = (
                out_ref[pl.ds(g * GROUP, GROUP), :] * pre
            )

        for k, d in enumerate(STEP_DISTS):
            rdma = descs[k]

            @pl.when(my_pos + d < N_DEV)
            def _():
                rdma.wait_send()

        @functools.partial(
            pl.run_scoped, second_barrier=pltpu.SemaphoreType.REGULAR
        )
        def _(second_barrier):
            for off in range(1, N_DEV):
                pl.semaphore_signal(
                    second_barrier, inc=1,
                    device_id=(lax.rem(my_pos + off, N_DEV),),
                    device_id_type=pl.DeviceIdType.MESH,
                )
            pl.semaphore_wait(second_barrier, N_DEV - 1)

    return pl.pallas_call(
        body,
        out_shape=jax.ShapeDtypeStruct((m, n), jnp.float32),
        in_specs=[pl.BlockSpec(memory_space=pltpu.VMEM)],
        out_specs=pl.BlockSpec(memory_space=pltpu.VMEM),
        scratch_shapes=[
            pltpu.VMEM((len(STEP_DISTS), n), jnp.float32),
            pltpu.VMEM((len(STEP_DISTS), n), jnp.float32),
            pltpu.SemaphoreType.DMA((len(STEP_DISTS),)),
            pltpu.SemaphoreType.DMA((len(STEP_DISTS),)),
        ],
        compiler_params=pltpu.CompilerParams(collective_id=0),
    )(x)


# device time: 17929 ns/iter; 1.6493x vs baseline; 1.1208x over previous
import functools

import jax
import jax.numpy as jnp
from jax import lax
from jax.experimental import pallas as pl
from jax.experimental.pallas import tpu as pltpu

N_DEV = 8
GROUP = 128
N_GROUPS = 2048 // GROUP


def kernel(x):
    m, n = x.shape

    def body(x_ref, out_ref, send_row, totals_buf, send_sems, recv_sems):
        my_pos = lax.axis_index("i")

        barrier_sem = pltpu.get_barrier_semaphore()
        for off in range(1, N_DEV):
            pl.semaphore_signal(
                barrier_sem, inc=1,
                device_id=(lax.rem(my_pos + off, N_DEV),),
                device_id_type=pl.DeviceIdType.MESH,
            )
        pl.semaphore_wait(barrier_sem, N_DEV - 1)

        ones_row = jnp.ones((1, n), jnp.float32)

        gts = []
        for g in range(N_GROUPS):
            u = x_ref[pl.ds(g * GROUP, GROUP), :]
            r = GROUP
            while r > 1:
                u = u[: r // 2] * u[r // 2 : r]
                r //= 2
            gts.append(u)
        gps = [ones_row]
        for g in range(1, N_GROUPS):
            gps.append(gps[g - 1] * gts[g - 1])
        send_row[...] = gps[-1] * gts[-1]

        descs = []
        for o in range(1, N_DEV):
            rdma = pltpu.make_async_remote_copy(
                src_ref=send_row,
                dst_ref=totals_buf.at[pl.ds(o, 1)],
                send_sem=send_sems.at[o],
                recv_sem=recv_sems.at[o],
                device_id=(lax.rem(my_pos + o, N_DEV),),
                device_id_type=pl.DeviceIdType.MESH,
            )
            descs.append(rdma)

            @pl.when(my_pos + o < N_DEV)
            def _():
                rdma.start()

        for g in range(N_GROUPS):
            v = x_ref[pl.ds(g * GROUP, GROUP), :]
            d = 1
            while d < GROUP:
                shifted = jnp.concatenate(
                    [jnp.ones((d, n), jnp.float32), v[: GROUP - d]], axis=0
                )
                v = v * shifted
                d *= 2
            out_ref[pl.ds(g * GROUP, GROUP), :] = v * gps[g]

        for o in range(1, N_DEV):
            rdma = descs[o - 1]

            @pl.when(o <= my_pos)
            def _():
                rdma.wait_recv()

        row = lax.broadcasted_iota(jnp.int32, (N_DEV, n), 0)
        mask = (row >= 1) & (row <= my_pos)
        t = jnp.where(mask, totals_buf[...], jnp.ones((N_DEV, n), jnp.float32))
        t = t[0:4] * t[4:8]
        t = t[0:2] * t[2:4]
        pre = t[0:1] * t[1:2]

        for g in range(N_GROUPS):
            out_ref[pl.ds(g * GROUP, GROUP), :] = (
                out_ref[pl.ds(g * GROUP, GROUP), :] * pre
            )

        for o in range(1, N_DEV):
            rdma = descs[o - 1]

            @pl.when(my_pos + o < N_DEV)
            def _():
                rdma.wait_send()

        @functools.partial(
            pl.run_scoped, second_barrier=pltpu.SemaphoreType.REGULAR
        )
        def _(second_barrier):
            for off in range(1, N_DEV):
                pl.semaphore_signal(
                    second_barrier, inc=1,
                    device_id=(lax.rem(my_pos + off, N_DEV),),
                    device_id_type=pl.DeviceIdType.MESH,
                )
            pl.semaphore_wait(second_barrier, N_DEV - 1)

    return pl.pallas_call(
        body,
        out_shape=jax.ShapeDtypeStruct((m, n), jnp.float32),
        in_specs=[pl.BlockSpec(memory_space=pltpu.VMEM)],
        out_specs=pl.BlockSpec(memory_space=pltpu.VMEM),
        scratch_shapes=[
            pltpu.VMEM((1, n), jnp.float32),
            pltpu.VMEM((N_DEV, n), jnp.float32),
            pltpu.SemaphoreType.DMA((N_DEV,)),
            pltpu.SemaphoreType.DMA((N_DEV,)),
        ],
        compiler_params=pltpu.CompilerParams(collective_id=0),
    )(x)
